# Initial kernel scaffold; baseline (speedup 1.0000x reference)
import jax
import jax.numpy as jnp
from jax.experimental import pallas as pl


def kernel(predicts, labels, sim_all, epoch, T, mu, eta):
    raise NotImplementedError("write your pallas kernel here")


# speedup vs baseline: 48.8406x; 48.8406x over previous
import jax
import jax.numpy as jnp
from jax import lax
from jax.experimental import pallas as pl
from jax.experimental.pallas import tpu as pltpu

_B = 256
_C = 100
_CP = 128
_EPS = 1e-6
_F32 = jnp.float32
_HI = lax.Precision.HIGHEST


def _body(params_ref, pred_ref, labc_ref, labr_ref, sim_ref, loss_ref, simb_ref):
    T = params_ref[0]
    mu = params_ref[1]
    eta = params_ref[2]
    epochf = params_ref[3]

    x = pred_ref[...] / T
    m = jnp.max(x, axis=1, keepdims=True)
    e = jnp.exp(x - m)
    s = jnp.sum(e, axis=1, keepdims=True)
    p = e / s

    logpe = jnp.log(p + _EPS)
    ent = jnp.sum(p * jnp.log(p + 1e-30), axis=1, keepdims=True)

    dn = (((1,), (1,)), ((), ()))
    K = ent - lax.dot_general(p, logpe, dn, precision=_HI,
                              preferred_element_type=_F32)
    K = K + _EPS * (1.0 + _C * _EPS)

    labc = labc_ref[...]
    labr = labr_ref[...]
    ioc = lax.broadcasted_iota(jnp.int32, (_B, _CP), 1)
    onehot = (labc == ioc).astype(_F32)

    G = lax.dot_general(p, onehot, dn, precision=_HI,
                        preferred_element_type=_F32)
    Gt = lax.dot_general(onehot, p, dn, precision=_HI,
                         preferred_element_type=_F32)
    d_col = jnp.sum(p * onehot, axis=1, keepdims=True)

    ia = lax.broadcasted_iota(jnp.int32, (_B, _B), 0)
    ib = lax.broadcasted_iota(jnp.int32, (_B, _B), 1)
    eye = (ia == ib).astype(_F32)
    d_row = jnp.sum(G * eye, axis=0, keepdims=True)

    term1 = d_col * jnp.log(d_col / (Gt + _EPS) + _EPS)
    term2 = G * jnp.log(G / (d_row + _EPS) + _EPS)
    S = term1 + term2

    R = lax.dot_general(onehot, sim_ref[...], (((1,), (0,)), ((), ())),
                        precision=_HI, preferred_element_type=_F32)
    simb = lax.dot_general(R, onehot, dn, precision=_HI,
                           preferred_element_type=_F32)

    m1 = jnp.max(p, axis=1, keepdims=True)
    idx1 = jnp.min(jnp.where(p == m1, ioc, _CP), axis=1, keepdims=True)
    p2 = jnp.where(ioc == idx1, -1.0, p)
    m2 = jnp.max(p2, axis=1, keepdims=True)
    idx2 = jnp.min(jnp.where(p2 == m2, ioc, _CP), axis=1, keepdims=True)
    maskf = (labc == idx1).astype(_F32)

    M1 = onehot * maskf
    M2 = (idx2 == ioc).astype(_F32)
    simb_ref[...] = lax.dot_general(M1, M2, (((0,), (0,)), ((), ())),
                                    precision=_HI,
                                    preferred_element_type=_F32)

    triu = (ib > ia).astype(_F32)
    same = (labc == labr).astype(_F32)
    same_t = triu * same
    diff_t = triu * (1.0 - same)

    IC_sum = jnp.sum(jnp.abs(K) * same_t)
    simw = jnp.where(epochf == 0.0, 1.0, simb)
    ISC_sum = jnp.sum(jnp.abs(S * simw) * diff_t)
    same_count = jnp.sum(same_t)
    diff_count = jnp.sum(diff_t)

    IC = jnp.where(same_count != 0.0, IC_sum / same_count, IC_sum)
    ISC = jnp.where(diff_count != 0.0, ISC_sum / diff_count, ISC_sum)
    ISC = jnp.where(ISC != 0.0, 1.0 / (ISC + _EPS) * mu, ISC)
    loss_ref[...] = jnp.broadcast_to(IC * eta + ISC, (1, 1))


def kernel(predicts, labels, sim_all, epoch, T, mu, eta):
    B, C = predicts.shape
    pred_pad = jnp.full((B, _CP), -1e30, dtype=_F32).at[:, :C].set(
        predicts.astype(_F32))
    sim_pad = jnp.zeros((_CP, _CP), dtype=_F32).at[:C, :C].set(
        sim_all.astype(_F32))
    labc = labels.astype(jnp.int32).reshape(B, 1)
    labr = labels.astype(jnp.int32).reshape(1, B)
    params = jnp.stack([
        jnp.asarray(T, _F32), jnp.asarray(mu, _F32),
        jnp.asarray(eta, _F32), jnp.asarray(epoch, _F32)])

    loss, simb = pl.pallas_call(
        _body,
        in_specs=[
            pl.BlockSpec(memory_space=pltpu.SMEM),
            pl.BlockSpec(memory_space=pltpu.VMEM),
            pl.BlockSpec(memory_space=pltpu.VMEM),
            pl.BlockSpec(memory_space=pltpu.VMEM),
            pl.BlockSpec(memory_space=pltpu.VMEM),
        ],
        out_specs=[
            pl.BlockSpec(memory_space=pltpu.VMEM),
            pl.BlockSpec(memory_space=pltpu.VMEM),
        ],
        out_shape=[
            jax.ShapeDtypeStruct((1, 1), _F32),
            jax.ShapeDtypeStruct((_CP, _CP), _F32),
        ],
    )(params, pred_pad, labc, labr, sim_pad)

    return loss.reshape(()), simb[:C, :C]


# speedup vs baseline: 70.8007x; 1.4496x over previous
import jax
import jax.numpy as jnp
from jax import lax
from jax.experimental import pallas as pl
from jax.experimental.pallas import tpu as pltpu

_B = 256
_C = 100
_EPS = 1e-6
_F32 = jnp.float32
_HI = lax.Precision.HIGHEST
_DN = (((1,), (1,)), ((), ()))


def _body(params_ref, pred_ref, labr_ref, sim_ref, loss_ref, simb_ref):
    T = params_ref[0]
    mu = params_ref[1]
    eta = params_ref[2]
    epochf = params_ref[3]

    x = pred_ref[...] / T
    m = jnp.max(x, axis=1, keepdims=True)
    e = jnp.exp(x - m)
    s = jnp.sum(e, axis=1, keepdims=True)
    p = e / s

    logpe = jnp.log(p + _EPS)
    ent = jnp.sum(p * jnp.log(p + 1e-30), axis=1, keepdims=True)

    K = ent - lax.dot_general(p, logpe, _DN, precision=_HI,
                              preferred_element_type=_F32)
    K = K + _EPS * (1.0 + _C * _EPS)

    ia = lax.broadcasted_iota(jnp.int32, (_B, _B), 0)
    ib = lax.broadcasted_iota(jnp.int32, (_B, _B), 1)
    eyef = (ia == ib).astype(_F32)

    labrf = labr_ref[...].astype(_F32)
    labcf = jnp.sum(eyef * labrf, axis=1, keepdims=True)
    iocf = lax.broadcasted_iota(jnp.int32, (_B, _C), 1).astype(_F32)
    onehot = (labcf == iocf).astype(_F32)

    G = lax.dot_general(p, onehot, _DN, precision=_HI,
                        preferred_element_type=_F32)
    Gt = lax.dot_general(onehot, p, _DN, precision=_HI,
                         preferred_element_type=_F32)
    d_col = jnp.sum(p * onehot, axis=1, keepdims=True)
    d_row = jnp.sum(G * eyef, axis=0, keepdims=True)

    term1 = d_col * jnp.log(d_col / (Gt + _EPS) + _EPS)
    term2 = G * jnp.log(G / (d_row + _EPS) + _EPS)
    S = term1 + term2

    R = lax.dot_general(onehot, sim_ref[...], (((1,), (0,)), ((), ())),
                        precision=_HI, preferred_element_type=_F32)
    simb = lax.dot_general(R, onehot, _DN, precision=_HI,
                           preferred_element_type=_F32)

    m1 = jnp.max(p, axis=1, keepdims=True)
    big = float(_C + 28)
    idx1 = jnp.min(jnp.where(p == m1, iocf, big), axis=1, keepdims=True)
    p2 = jnp.where(iocf == idx1, -1.0, p)
    m2 = jnp.max(p2, axis=1, keepdims=True)
    idx2 = jnp.min(jnp.where(p2 == m2, iocf, big), axis=1, keepdims=True)
    maskf = (labcf == idx1).astype(_F32)

    M1 = onehot * maskf
    M2 = (idx2 == iocf).astype(_F32)
    simb_ref[...] = lax.dot_general(M1, M2, (((0,), (0,)), ((), ())),
                                    precision=_HI,
                                    preferred_element_type=_F32)

    triu = (ib > ia).astype(_F32)
    same = (labcf == labrf).astype(_F32)
    same_t = triu * same
    diff_t = triu * (1.0 - same)

    IC_sum = jnp.sum(jnp.abs(K) * same_t)
    simw = jnp.where(epochf == 0.0, 1.0, simb)
    ISC_sum = jnp.sum(jnp.abs(S * simw) * diff_t)
    same_count = jnp.sum(same_t)
    diff_count = jnp.sum(diff_t)

    IC = jnp.where(same_count != 0.0, IC_sum / same_count, IC_sum)
    ISC = jnp.where(diff_count != 0.0, ISC_sum / diff_count, ISC_sum)
    ISC = jnp.where(ISC != 0.0, 1.0 / (ISC + _EPS) * mu, ISC)
    loss_ref[...] = jnp.broadcast_to(IC * eta + ISC, (1, 1))


def kernel(predicts, labels, sim_all, epoch, T, mu, eta):
    B, C = predicts.shape
    labr = labels.astype(jnp.int32).reshape(1, B)
    params = jnp.stack([
        jnp.asarray(T, _F32), jnp.asarray(mu, _F32),
        jnp.asarray(eta, _F32), jnp.asarray(epoch, _F32)])

    loss, simb = pl.pallas_call(
        _body,
        in_specs=[
            pl.BlockSpec(memory_space=pltpu.SMEM),
            pl.BlockSpec(memory_space=pltpu.VMEM),
            pl.BlockSpec(memory_space=pltpu.VMEM),
            pl.BlockSpec(memory_space=pltpu.VMEM),
        ],
        out_specs=[
            pl.BlockSpec(memory_space=pltpu.VMEM),
            pl.BlockSpec(memory_space=pltpu.VMEM),
        ],
        out_shape=[
            jax.ShapeDtypeStruct((1, 1), _F32),
            jax.ShapeDtypeStruct((C, C), _F32),
        ],
    )(params, predicts.astype(_F32), labr, sim_all.astype(_F32))

    return loss.reshape(()), simb


# speedup vs baseline: 76.5327x; 1.0810x over previous
import jax
import jax.numpy as jnp
from jax import lax
from jax.experimental import pallas as pl
from jax.experimental.pallas import tpu as pltpu

_B = 256
_C = 100
_EPS = 1e-6
_F32 = jnp.float32
_HI = lax.Precision.HIGHEST
_DN = (((1,), (1,)), ((), ()))


def _body(t_ref, mu_ref, eta_ref, ep_ref, pred_ref, labr_ref, sim_ref,
          loss_ref, simb_ref):
    T = t_ref[0, 0]
    mu = mu_ref[0, 0]
    eta = eta_ref[0, 0]
    epochi = ep_ref[0, 0]

    x = pred_ref[...] / T
    m = jnp.max(x, axis=1, keepdims=True)
    e = jnp.exp(x - m)
    s = jnp.sum(e, axis=1, keepdims=True)
    p = e / s

    logpe = jnp.log(p + _EPS)
    ent = jnp.sum(p * jnp.log(p + 1e-30), axis=1, keepdims=True)

    K = ent - lax.dot_general(p, logpe, _DN, precision=_HI,
                              preferred_element_type=_F32)
    K = K + _EPS * (1.0 + _C * _EPS)

    ia = lax.broadcasted_iota(jnp.int32, (_B, _B), 0)
    ib = lax.broadcasted_iota(jnp.int32, (_B, _B), 1)
    eyef = (ia == ib).astype(_F32)

    labrf = labr_ref[...].astype(_F32)
    labcf = jnp.sum(eyef * labrf, axis=1, keepdims=True)
    iocf = lax.broadcasted_iota(jnp.int32, (_B, _C), 1).astype(_F32)
    onehot = (labcf == iocf).astype(_F32)

    G = lax.dot_general(p, onehot, _DN, precision=_HI,
                        preferred_element_type=_F32)
    Gt = lax.dot_general(onehot, p, _DN, precision=_HI,
                         preferred_element_type=_F32)
    d_col = jnp.sum(p * onehot, axis=1, keepdims=True)
    d_row = jnp.sum(G * eyef, axis=0, keepdims=True)

    term1 = d_col * jnp.log(d_col / (Gt + _EPS) + _EPS)
    term2 = G * jnp.log(G / (d_row + _EPS) + _EPS)
    S = term1 + term2

    R = lax.dot_general(onehot, sim_ref[...], (((1,), (0,)), ((), ())),
                        precision=_HI, preferred_element_type=_F32)
    simb = lax.dot_general(R, onehot, _DN, precision=_HI,
                           preferred_element_type=_F32)

    m1 = jnp.max(p, axis=1, keepdims=True)
    big = float(_C + 28)
    idx1 = jnp.min(jnp.where(p == m1, iocf, big), axis=1, keepdims=True)
    p2 = jnp.where(iocf == idx1, -1.0, p)
    m2 = jnp.max(p2, axis=1, keepdims=True)
    idx2 = jnp.min(jnp.where(p2 == m2, iocf, big), axis=1, keepdims=True)
    maskf = (labcf == idx1).astype(_F32)

    M1 = onehot * maskf
    M2 = (idx2 == iocf).astype(_F32)
    simb_ref[...] = lax.dot_general(M1, M2, (((0,), (0,)), ((), ())),
                                    precision=_HI,
                                    preferred_element_type=_F32)

    triu = (ib > ia).astype(_F32)
    same = (labcf == labrf).astype(_F32)
    same_t = triu * same
    diff_t = triu * (1.0 - same)

    IC_sum = jnp.sum(jnp.abs(K) * same_t)
    simw = jnp.where(epochi == 0, 1.0, simb)
    ISC_sum = jnp.sum(jnp.abs(S * simw) * diff_t)
    same_count = jnp.sum(same_t)
    diff_count = jnp.sum(diff_t)

    IC = jnp.where(same_count != 0.0, IC_sum / same_count, IC_sum)
    ISC = jnp.where(diff_count != 0.0, ISC_sum / diff_count, ISC_sum)
    ISC = jnp.where(ISC != 0.0, 1.0 / (ISC + _EPS) * mu, ISC)
    loss_ref[...] = jnp.broadcast_to(IC * eta + ISC, (1, 1))


def kernel(predicts, labels, sim_all, epoch, T, mu, eta):
    B, C = predicts.shape
    labr = labels.astype(jnp.int32).reshape(1, B)
    t_a = jnp.asarray(T, _F32).reshape(1, 1)
    mu_a = jnp.asarray(mu, _F32).reshape(1, 1)
    eta_a = jnp.asarray(eta, _F32).reshape(1, 1)
    ep_a = jnp.asarray(epoch, jnp.int32).reshape(1, 1)

    loss, simb = pl.pallas_call(
        _body,
        in_specs=[
            pl.BlockSpec(memory_space=pltpu.SMEM),
            pl.BlockSpec(memory_space=pltpu.SMEM),
            pl.BlockSpec(memory_space=pltpu.SMEM),
            pl.BlockSpec(memory_space=pltpu.SMEM),
            pl.BlockSpec(memory_space=pltpu.VMEM),
            pl.BlockSpec(memory_space=pltpu.VMEM),
            pl.BlockSpec(memory_space=pltpu.VMEM),
        ],
        out_specs=[
            pl.BlockSpec(memory_space=pltpu.VMEM),
            pl.BlockSpec(memory_space=pltpu.VMEM),
        ],
        out_shape=[
            jax.ShapeDtypeStruct((1, 1), _F32),
            jax.ShapeDtypeStruct((C, C), _F32),
        ],
    )(t_a, mu_a, eta_a, ep_a, predicts.astype(_F32), labr,
      sim_all.astype(_F32))

    return loss.reshape(()), simb
